# Initial kernel scaffold; baseline (speedup 1.0000x reference)
import jax
import jax.numpy as jnp
from jax.experimental import pallas as pl


def kernel(z_cur, z_res, codebook, log_param_q_scalar_q, flg_train, flg_quant_det):
    raise NotImplementedError("write your pallas kernel here")


# speedup vs baseline: 1.5249x; 1.5249x over previous
import functools

import jax
import jax.numpy as jnp
from jax import lax
from jax.experimental import pallas as pl
from jax.experimental.pallas import tpu as pltpu

B, T, D, K = 4, 1024, 256, 8192
N = B * T
TB = 256
NSTEPS = N // TB


def _vq_body(prec_ref, z_ref, zcur_ref, cb_ref,
             zcur_out, zres_out, zq_out, kld_out, perp_out,
             probs_acc, plogp_acc):
    i = pl.program_id(0)

    @pl.when(i == 0)
    def _init():
        probs_acc[...] = jnp.zeros_like(probs_acc)
        plogp_acc[...] = jnp.zeros_like(plogp_acc)

    prec = prec_ref[0, 0]
    z = z_ref[...]
    c = cb_ref[...]

    zc = lax.dot_general(z, c, (((1,), (1,)), ((), ())),
                         preferred_element_type=jnp.float32)
    z2 = jnp.sum(z * z, axis=1, keepdims=True)
    c2 = jnp.sum(c * c, axis=1)
    dist = z2 - 2.0 * zc + c2[None, :]
    logits = -prec * dist

    m = jnp.max(logits, axis=1, keepdims=True)
    iota = lax.broadcasted_iota(jnp.int32, (TB, K), 1)
    idx = jnp.min(jnp.where(logits == m, iota, K), axis=1)

    e = jnp.exp(logits - m)
    s = jnp.sum(e, axis=1, keepdims=True)
    p = e / s
    lp = (logits - m) - jnp.log(s)
    row_kld = jnp.sum(p * (lp + jnp.log(float(K))), axis=1)

    plogp_acc[...] = plogp_acc[...] + jnp.sum(row_kld)
    probs_acc[...] += jnp.sum(p, axis=0, keepdims=True)

    onehot = (iota == idx[:, None]).astype(jnp.float32)
    zq = lax.dot_general(onehot, c, (((1,), (0,)), ((), ())),
                         preferred_element_type=jnp.float32)
    zq_out[...] = zq
    zcur_out[...] = zcur_ref[...] + zq
    zres_out[...] = z - zq

    @pl.when(i == NSTEPS - 1)
    def _fin():
        avg = probs_acc[...] / float(N)
        kld_out[...] = plogp_acc[...] / float(N)
        perp_out[...] = jnp.zeros_like(perp_out) + jnp.exp(
            -jnp.sum(avg * jnp.log(avg + 1e-7)))


@functools.partial(jax.jit, static_argnames=())
def _vq_fused(z_res, z_cur, codebook, prec):
    grid = (NSTEPS,)
    out = pl.pallas_call(
        _vq_body,
        grid=grid,
        in_specs=[
            pl.BlockSpec(memory_space=pltpu.SMEM),
            pl.BlockSpec((TB, D), lambda i: (i, 0)),
            pl.BlockSpec((TB, D), lambda i: (i, 0)),
            pl.BlockSpec((K, D), lambda i: (0, 0)),
        ],
        out_specs=[
            pl.BlockSpec((TB, D), lambda i: (i, 0)),
            pl.BlockSpec((TB, D), lambda i: (i, 0)),
            pl.BlockSpec((TB, D), lambda i: (i, 0)),
            pl.BlockSpec((1, 1), lambda i: (0, 0)),
            pl.BlockSpec((1, 1), lambda i: (0, 0)),
        ],
        out_shape=[
            jax.ShapeDtypeStruct((N, D), jnp.float32),
            jax.ShapeDtypeStruct((N, D), jnp.float32),
            jax.ShapeDtypeStruct((N, D), jnp.float32),
            jax.ShapeDtypeStruct((1, 1), jnp.float32),
            jax.ShapeDtypeStruct((1, 1), jnp.float32),
        ],
        scratch_shapes=[
            pltpu.VMEM((1, K), jnp.float32),
            pltpu.VMEM((1, 1), jnp.float32),
        ],
        compiler_params=pltpu.CompilerParams(
            dimension_semantics=("arbitrary",),
        ),
    )(prec, z_res, z_cur, codebook)
    return out


def kernel(z_cur, z_res, codebook, log_param_q_scalar_q, flg_train, flg_quant_det):
    del flg_train, flg_quant_det
    prec = (0.5 / jnp.exp(log_param_q_scalar_q)).reshape(1, 1).astype(jnp.float32)
    zr = z_res.reshape(N, D)
    zc_ = z_cur.reshape(N, D)
    z_cur_new, z_res_new, z_q, kld, perp = _vq_fused(zr, zc_, codebook, prec)
    return (z_cur_new.reshape(B, T, D),
            z_res_new.reshape(B, T, D),
            z_q.reshape(B, T, D),
            kld[0, 0],
            perp[0, 0])


# speedup vs baseline: 1.5632x; 1.0251x over previous
import functools

import jax
import jax.numpy as jnp
from jax import lax
from jax.experimental import pallas as pl
from jax.experimental.pallas import tpu as pltpu

B, T, D, K = 4, 1024, 256, 8192
N = B * T
TB = 256
NSTEPS = N // TB


def _vq_body(prec_ref, z_ref, zcur_ref, cb_ref, cb16_ref,
             zcur_out, zres_out, zq_out, kld_out, perp_out,
             probs_acc, plogp_acc, c2_acc):
    i = pl.program_id(0)

    @pl.when(i == 0)
    def _init():
        probs_acc[...] = jnp.zeros_like(probs_acc)
        plogp_acc[...] = jnp.zeros_like(plogp_acc)
        c = cb_ref[...]
        c2_acc[...] = jnp.sum(c * c, axis=1)[None, :]

    prec = prec_ref[0, 0]
    z = z_ref[...]
    c = cb_ref[...]

    zc = lax.dot_general(z, c, (((1,), (1,)), ((), ())),
                         preferred_element_type=jnp.float32)
    z2 = jnp.sum(z * z, axis=1, keepdims=True)
    dist = z2 - 2.0 * zc + c2_acc[...]
    logits = -prec * dist

    m = jnp.max(logits, axis=1, keepdims=True)
    iota = lax.broadcasted_iota(jnp.int32, (TB, K), 1)
    idx = jnp.min(jnp.where(logits == m, iota, K), axis=1)

    t = logits - m
    e = jnp.exp(t)
    s = jnp.sum(e, axis=1, keepdims=True)
    rinv = 1.0 / s
    p = e * rinv
    row_kld = (jnp.sum(e * t, axis=1, keepdims=True) * rinv
               - jnp.log(s) + jnp.log(float(K)))

    plogp_acc[...] = plogp_acc[...] + jnp.sum(row_kld)
    probs_acc[...] += jnp.sum(p, axis=0, keepdims=True)

    onehot = (iota == idx[:, None]).astype(jnp.bfloat16)
    zq = lax.dot_general(onehot, cb16_ref[...], (((1,), (0,)), ((), ())),
                         preferred_element_type=jnp.float32)
    zq_out[...] = zq
    zcur_out[...] = zcur_ref[...] + zq
    zres_out[...] = z - zq

    @pl.when(i == NSTEPS - 1)
    def _fin():
        avg = probs_acc[...] / float(N)
        kld_out[...] = plogp_acc[...] / float(N)
        perp_out[...] = jnp.zeros_like(perp_out) + jnp.exp(
            -jnp.sum(avg * jnp.log(avg + 1e-7)))


@functools.partial(jax.jit, static_argnames=())
def _vq_fused(z_res, z_cur, codebook, cb16, prec):
    grid = (NSTEPS,)
    out = pl.pallas_call(
        _vq_body,
        grid=grid,
        in_specs=[
            pl.BlockSpec(memory_space=pltpu.SMEM),
            pl.BlockSpec((TB, D), lambda i: (i, 0)),
            pl.BlockSpec((TB, D), lambda i: (i, 0)),
            pl.BlockSpec((K, D), lambda i: (0, 0)),
            pl.BlockSpec((K, D), lambda i: (0, 0)),
        ],
        out_specs=[
            pl.BlockSpec((TB, D), lambda i: (i, 0)),
            pl.BlockSpec((TB, D), lambda i: (i, 0)),
            pl.BlockSpec((TB, D), lambda i: (i, 0)),
            pl.BlockSpec((1, 1), lambda i: (0, 0)),
            pl.BlockSpec((1, 1), lambda i: (0, 0)),
        ],
        out_shape=[
            jax.ShapeDtypeStruct((N, D), jnp.float32),
            jax.ShapeDtypeStruct((N, D), jnp.float32),
            jax.ShapeDtypeStruct((N, D), jnp.float32),
            jax.ShapeDtypeStruct((1, 1), jnp.float32),
            jax.ShapeDtypeStruct((1, 1), jnp.float32),
        ],
        scratch_shapes=[
            pltpu.VMEM((1, K), jnp.float32),
            pltpu.VMEM((1, 1), jnp.float32),
            pltpu.VMEM((1, K), jnp.float32),
        ],
        compiler_params=pltpu.CompilerParams(
            dimension_semantics=("arbitrary",),
        ),
    )(prec, z_res, z_cur, codebook, cb16)
    return out


def kernel(z_cur, z_res, codebook, log_param_q_scalar_q, flg_train, flg_quant_det):
    del flg_train, flg_quant_det
    prec = (0.5 / jnp.exp(log_param_q_scalar_q)).reshape(1, 1).astype(jnp.float32)
    zr = z_res.reshape(N, D)
    zc_ = z_cur.reshape(N, D)
    cb16 = codebook.astype(jnp.bfloat16)
    z_cur_new, z_res_new, z_q, kld, perp = _vq_fused(zr, zc_, codebook, cb16, prec)
    return (z_cur_new.reshape(B, T, D),
            z_res_new.reshape(B, T, D),
            z_q.reshape(B, T, D),
            kld[0, 0],
            perp[0, 0])


# speedup vs baseline: 1.6203x; 1.0366x over previous
import functools

import jax
import jax.numpy as jnp
from jax import lax
from jax.experimental import pallas as pl
from jax.experimental.pallas import tpu as pltpu

B, T, D, K = 4, 1024, 256, 8192
N = B * T
TB = 256
NSTEPS = N // TB


def _vq_body(prec_ref, z_ref, zcur_ref, cb_ref, cb16_ref,
             zcur_out, zres_out, zq_out, kld_out, perp_out,
             probs_acc, plogp_acc, c2_acc):
    i = pl.program_id(0)

    @pl.when(i == 0)
    def _init():
        probs_acc[...] = jnp.zeros_like(probs_acc)
        plogp_acc[...] = jnp.zeros_like(plogp_acc)
        c = cb_ref[...]
        c2_acc[...] = jnp.sum(c * c, axis=1)[None, :]

    prec = prec_ref[0, 0]
    z = z_ref[...]
    c = cb_ref[...]
    ones_d = jnp.ones((D, 1), jnp.float32)
    ones_k = jnp.ones((K, 1), jnp.float32)

    zc = lax.dot_general(z, c, (((1,), (1,)), ((), ())),
                         preferred_element_type=jnp.float32)
    z2 = lax.dot_general(z * z, ones_d, (((1,), (0,)), ((), ())),
                         preferred_element_type=jnp.float32)
    dist = z2 - 2.0 * zc + c2_acc[...]
    logits = -prec * dist

    m = jnp.max(logits, axis=1, keepdims=True)
    iota = lax.broadcasted_iota(jnp.int32, (TB, K), 1)
    idx = jnp.min(jnp.where(logits == m, iota, K), axis=1)

    t = logits - m
    e = jnp.exp(t)
    et = e * t
    s = lax.dot_general(e, ones_k, (((1,), (0,)), ((), ())),
                        preferred_element_type=jnp.float32)
    set_ = lax.dot_general(et, ones_k, (((1,), (0,)), ((), ())),
                           preferred_element_type=jnp.float32)
    rinv = 1.0 / s
    row_kld = set_ * rinv - jnp.log(s) + jnp.log(float(K))

    plogp_acc[...] = plogp_acc[...] + jnp.sum(row_kld)
    probs_acc[...] += lax.dot_general(rinv, e, (((0,), (0,)), ((), ())),
                                      preferred_element_type=jnp.float32)

    onehot = (iota == idx[:, None]).astype(jnp.bfloat16)
    zq = lax.dot_general(onehot, cb16_ref[...], (((1,), (0,)), ((), ())),
                         preferred_element_type=jnp.float32)
    zq_out[...] = zq
    zcur_out[...] = zcur_ref[...] + zq
    zres_out[...] = z - zq

    @pl.when(i == NSTEPS - 1)
    def _fin():
        avg = probs_acc[...] / float(N)
        kld_out[...] = plogp_acc[...] / float(N)
        perp_out[...] = jnp.zeros_like(perp_out) + jnp.exp(
            -jnp.sum(avg * jnp.log(avg + 1e-7)))


@functools.partial(jax.jit, static_argnames=())
def _vq_fused(z_res, z_cur, codebook, cb16, prec):
    grid = (NSTEPS,)
    out = pl.pallas_call(
        _vq_body,
        grid=grid,
        in_specs=[
            pl.BlockSpec(memory_space=pltpu.SMEM),
            pl.BlockSpec((TB, D), lambda i: (i, 0)),
            pl.BlockSpec((TB, D), lambda i: (i, 0)),
            pl.BlockSpec((K, D), lambda i: (0, 0)),
            pl.BlockSpec((K, D), lambda i: (0, 0)),
        ],
        out_specs=[
            pl.BlockSpec((TB, D), lambda i: (i, 0)),
            pl.BlockSpec((TB, D), lambda i: (i, 0)),
            pl.BlockSpec((TB, D), lambda i: (i, 0)),
            pl.BlockSpec((1, 1), lambda i: (0, 0)),
            pl.BlockSpec((1, 1), lambda i: (0, 0)),
        ],
        out_shape=[
            jax.ShapeDtypeStruct((N, D), jnp.float32),
            jax.ShapeDtypeStruct((N, D), jnp.float32),
            jax.ShapeDtypeStruct((N, D), jnp.float32),
            jax.ShapeDtypeStruct((1, 1), jnp.float32),
            jax.ShapeDtypeStruct((1, 1), jnp.float32),
        ],
        scratch_shapes=[
            pltpu.VMEM((1, K), jnp.float32),
            pltpu.VMEM((1, 1), jnp.float32),
            pltpu.VMEM((1, K), jnp.float32),
        ],
        compiler_params=pltpu.CompilerParams(
            dimension_semantics=("arbitrary",),
        ),
    )(prec, z_res, z_cur, codebook, cb16)
    return out


def kernel(z_cur, z_res, codebook, log_param_q_scalar_q, flg_train, flg_quant_det):
    del flg_train, flg_quant_det
    prec = (0.5 / jnp.exp(log_param_q_scalar_q)).reshape(1, 1).astype(jnp.float32)
    zr = z_res.reshape(N, D)
    zc_ = z_cur.reshape(N, D)
    cb16 = codebook.astype(jnp.bfloat16)
    z_cur_new, z_res_new, z_q, kld, perp = _vq_fused(zr, zc_, codebook, cb16, prec)
    return (z_cur_new.reshape(B, T, D),
            z_res_new.reshape(B, T, D),
            z_q.reshape(B, T, D),
            kld[0, 0],
            perp[0, 0])
